# baseline (speedup 1.0000x reference)
import functools

import jax
import jax.numpy as jnp
from jax import lax
from jax.experimental import pallas as pl
from jax.experimental.pallas import tpu as pltpu
from jax.experimental.pallas import tpu_sc as plsc

B = 16384
NF = 26
NCONT = 13
D = 32
FT = NF + NCONT
NW = 32
RW = B // NW
CH = 128
NCH = RW // CH


def _body(xct_hbm, xnum_hbm, table_hbm, w_hbm, b_hbm, out_hbm,
          idx_v, rows_v, xnum_v, w_v, b_v, num_v, gsem):
    cid = lax.axis_index("c")
    sid = lax.axis_index("s")
    wid = sid * 2 + cid
    b0 = wid * RW

    pltpu.sync_copy(xnum_hbm.at[pl.ds(b0, RW)], xnum_v)
    pltpu.sync_copy(w_hbm, w_v)
    pltpu.sync_copy(b_hbm, b_v)

    for f in range(NF):
        pltpu.sync_copy(xct_hbm.at[f, pl.ds(wid * NCH, NCH)], idx_v)
        for j in range(NCH):
            pltpu.async_copy(
                table_hbm.at[idx_v.at[j]],
                rows_v.at[pl.ds(j * CH, CH)],
                gsem,
            ).wait()
        pltpu.sync_copy(rows_v, out_hbm.at[pl.ds(b0, RW), f])

    for n in range(NCONT):
        w0 = w_v[n, 0]
        w1 = w_v[n, 1]
        a0 = b_v[n, 0]
        a1 = b_v[n, 1]

        ncol = jnp.full((16,), n, jnp.int32)

        def row(i, _, n=n, w0=w0, w1=w1, a0=a0, a1=a1, ncol=ncol):
            s = plsc.load_gather(xnum_v, [jnp.full((16,), i, jnp.int32), ncol])
            num_v[i, pl.ds(0, 16)] = s * w0 + a0
            num_v[i, pl.ds(16, 16)] = s * w1 + a1
            return 0

        lax.fori_loop(0, RW, row, 0)
        pltpu.sync_copy(num_v, out_hbm.at[pl.ds(b0, RW), NF + n])


_embed = functools.partial(
    pl.kernel,
    out_type=jax.ShapeDtypeStruct((B, FT, D), jnp.float32),
    mesh=plsc.VectorSubcoreMesh(core_axis_name="c", subcore_axis_name="s"),
    compiler_params=pltpu.CompilerParams(
        use_tc_tiling_on_sc=False, needs_layout_passes=False
    ),
    scratch_types=[
        pltpu.VMEM((NCH, CH), jnp.int32),
        pltpu.VMEM((RW, D), jnp.float32),
        pltpu.VMEM((RW, NCONT), jnp.float32),
        pltpu.VMEM((NCONT, 2, 16), jnp.float32),
        pltpu.VMEM((NCONT, 2, 16), jnp.float32),
        pltpu.VMEM((RW, D), jnp.float32),
        pltpu.SemaphoreType.DMA,
    ],
)(_body)


def kernel(x_categ, x_numer, embed_table, num_weights, num_biases):
    xct = x_categ.astype(jnp.int32).T.reshape(NF, NW * NCH, CH)
    w = num_weights.reshape(NCONT, 2, 16)
    b = num_biases.reshape(NCONT, 2, 16)
    return _embed(xct, x_numer, embed_table, w, b)


# speedup vs baseline: 1.0535x; 1.0535x over previous
import functools

import jax
import jax.numpy as jnp
from jax import lax
from jax.experimental import pallas as pl
from jax.experimental.pallas import tpu as pltpu
from jax.experimental.pallas import tpu_sc as plsc

B = 16384
NF = 26
NCONT = 13
D = 32
FT = NF + NCONT
NW = 32
RW = B // NW
CH = 128
NCH = RW // CH


def _body(xct_hbm, xnum_hbm, table_hbm, w_hbm, b_hbm, out_hbm,
          idx_v, rows_v, xnum_v, w_v, b_v, num_v,
          isem0, isem1, gsem0, gsem1, wsem0, wsem1, wsem2, nsem0, nsem1):
    isem = (isem0, isem1)
    gsem = (gsem0, gsem1)
    wsem = (wsem0, wsem1, wsem2)
    nsem = (nsem0, nsem1)

    cid = lax.axis_index("c")
    sid = lax.axis_index("s")
    wid = sid * 2 + cid
    b0 = wid * RW

    pltpu.sync_copy(xnum_hbm.at[pl.ds(b0, RW)], xnum_v)
    pltpu.sync_copy(w_hbm, w_v)
    pltpu.sync_copy(b_hbm, b_v)

    def load_idx(f):
        return pltpu.async_copy(
            xct_hbm.at[f, pl.ds(wid * NCH, NCH)], idx_v.at[f % 2], isem[f % 2])

    def fire_gathers(f):
        return [
            pltpu.async_copy(
                table_hbm.at[idx_v.at[f % 2, j]],
                rows_v.at[f % 3, pl.ds(j * CH, CH)],
                gsem[f % 2],
            )
            for j in range(NCH)
        ]

    def fire_write(f):
        return pltpu.async_copy(
            rows_v.at[f % 3], out_hbm.at[pl.ds(b0, RW), f], wsem[f % 3])

    load_idx(0).wait()
    gathers = fire_gathers(0)
    iloads = {1: load_idx(1)}

    nwrites = {}
    for n in range(NCONT):
        w0 = w_v[n, 0]
        w1 = w_v[n, 1]
        a0 = b_v[n, 0]
        a1 = b_v[n, 1]
        ncol = jnp.full((16,), n, jnp.int32)
        if n >= 2:
            nwrites[n - 2].wait()

        def row(i, _, w0=w0, w1=w1, a0=a0, a1=a1, ncol=ncol, p=n % 2):
            s = plsc.load_gather(xnum_v, [jnp.full((16,), i, jnp.int32), ncol])
            num_v[p, i, pl.ds(0, 16)] = s * w0 + a0
            num_v[p, i, pl.ds(16, 16)] = s * w1 + a1
            return 0

        lax.fori_loop(0, RW, row, 0)
        nwrites[n] = pltpu.async_copy(
            num_v.at[n % 2], out_hbm.at[pl.ds(b0, RW), NF + n], nsem[n % 2])

    writes = {}
    for f in range(1, NF):
        iloads[f].wait()
        if f >= 3:
            writes[f - 3].wait()
        prev = gathers
        gathers = fire_gathers(f)
        for c in prev:
            c.wait()
        writes[f - 1] = fire_write(f - 1)
        if f + 1 < NF:
            iloads[f + 1] = load_idx(f + 1)

    for c in gathers:
        c.wait()
    writes[NF - 1] = fire_write(NF - 1)
    for f in (NF - 3, NF - 2, NF - 1):
        writes[f].wait()
    nwrites[NCONT - 2].wait()
    nwrites[NCONT - 1].wait()


_embed = functools.partial(
    pl.kernel,
    out_type=jax.ShapeDtypeStruct((B, FT, D), jnp.float32),
    mesh=plsc.VectorSubcoreMesh(core_axis_name="c", subcore_axis_name="s"),
    compiler_params=pltpu.CompilerParams(
        use_tc_tiling_on_sc=False, needs_layout_passes=False
    ),
    scratch_types=[
        pltpu.VMEM((2, NCH, CH), jnp.int32),
        pltpu.VMEM((3, RW, D), jnp.float32),
        pltpu.VMEM((RW, NCONT), jnp.float32),
        pltpu.VMEM((NCONT, 2, 16), jnp.float32),
        pltpu.VMEM((NCONT, 2, 16), jnp.float32),
        pltpu.VMEM((2, RW, D), jnp.float32),
    ] + [pltpu.SemaphoreType.DMA] * 9,
)(_body)


def kernel(x_categ, x_numer, embed_table, num_weights, num_biases):
    xct = x_categ.astype(jnp.int32).T.reshape(NF, NW * NCH, CH)
    w = num_weights.reshape(NCONT, 2, 16)
    b = num_biases.reshape(NCONT, 2, 16)
    return _embed(xct, x_numer, embed_table, w, b)
